# baseline (speedup 1.0000x reference)
import functools

import jax
import jax.numpy as jnp
from jax import lax
from jax.experimental import pallas as pl
from jax.experimental.pallas import tpu as pltpu, tpu_sc as plsc

N = 10000
D = 128
H = 128
E = 320000

NC = 2
NS = 16
NW = NC * NS

CHUNK = 128
SUB = 8
BLOCK_E = CHUNK * SUB
EPW_BLOCKS = 10
EPW = BLOCK_E * EPW_BLOCKS
E_PAD = EPW * NW
ROWS_PER_BLOCK = BLOCK_E // CHUNK * SUB // SUB

N_PAD = 10240
NPT = N_PAD // NS

_f32 = jnp.float32
_i32 = jnp.int32

_MESH = plsc.VectorSubcoreMesh(
    core_axis_name="c", subcore_axis_name="s", num_cores=NC, num_subcores=NS)


def _zero_fill(vref, rows, width):
    zero = jnp.zeros((16,), _f32)

    def body(i, _):
        for q in range(width // 16):
            vref[i, pl.ds(q * 16, 16)] = zero
        return 0

    lax.fori_loop(0, rows, body, 0)


def _deg_body(dst1, dst2, dst3, ew1, ew2, ew3, out,
              sh1, sh2, sh3, idx2, ewb, zbuf):
    c = lax.axis_index("c")
    s = lax.axis_index("s")
    w = s * NC + c
    shared = (sh1, sh2, sh3)
    dsts = (dst1, dst2, dst3)
    ews = (ew1, ew2, ew3)

    zero = jnp.zeros((16,), _f32)
    for i in range(40):
        zbuf[pl.ds(16 * i, 16)] = zero
    for o in range(3):
        pltpu.sync_copy(zbuf, shared[o].at[pl.ds(640 * s, 640)])
    plsc.subcore_barrier()

    for o in range(3):
        def chunk(k, _, o=o):
            rb = w * (EPW // CHUNK) + k * SUB
            pltpu.sync_copy(dsts[o].at[pl.ds(rb, SUB)], idx2)
            pltpu.sync_copy(ews[o].at[pl.ds(rb * CHUNK, BLOCK_E)], ewb)
            for j in range(SUB):
                pltpu.sync_copy(ewb.at[pl.ds(j * CHUNK, CHUNK)],
                                shared[o].at[idx2.at[j]], add=True)
            return 0
        lax.fori_loop(0, EPW_BLOCKS, chunk, 0)
    plsc.subcore_barrier()

    for o in range(3):
        pltpu.sync_copy(shared[o].at[pl.ds(640 * s, 640)],
                        out.at[o, c, pl.ds(640 * s, 640)])


_deg_call = pl.kernel(
    _deg_body,
    out_type=jax.ShapeDtypeStruct((3, NC, N_PAD), _f32),
    mesh=_MESH,
    scratch_types=[
        pltpu.VMEM_SHARED((N_PAD,), _f32),
        pltpu.VMEM_SHARED((N_PAD,), _f32),
        pltpu.VMEM_SHARED((N_PAD,), _f32),
        pltpu.VMEM((SUB, CHUNK), _i32),
        pltpu.VMEM((BLOCK_E,), _f32),
        pltpu.VMEM((640,), _f32),
    ],
)


RCHUNK = 64
RING = 4
CPB = BLOCK_E // RCHUNK
BLK0 = 19
BLK1 = 1


def _row_body(hp, src3d, dst3d, ew, out, acc_sh, isrc, idst, ewb,
              b0, b1, b2, b3, gsem0, gsem1, gsem2, gsem3,
              ssem0, ssem1, ssem2, ssem3):
    c = lax.axis_index("c")
    s = lax.axis_index("s")
    w = s * NC + c
    bufs = (b0, b1, b2, b3)
    gsems = (gsem0, gsem1, gsem2, gsem3)
    ssems = (ssem0, ssem1, ssem2, ssem3)

    _zero_fill(b0, RCHUNK, H)
    base = NPT * s
    for k in range(NPT // RCHUNK):
        pltpu.sync_copy(b0, acc_sh.at[pl.ds(base + RCHUNK * k, RCHUNK)])
    plsc.subcore_barrier()

    def issue_gather(b, t):
        pltpu.async_copy(hp.at[isrc.at[t]], bufs[b], gsems[b])

    def wait_gather(b):
        pltpu.make_async_copy(hp.at[isrc.at[0]], bufs[b], gsems[b]).wait()

    def issue_scatter(b, t):
        pltpu.async_copy(bufs[b], acc_sh.at[idst.at[t]], ssems[b], add=True)

    def wait_scatter(b):
        pltpu.make_async_copy(bufs[b], acc_sh.at[idst.at[0]], ssems[b]).wait()

    def scale(b, t):
        def grp(g16, _):
            ewv = ewb[pl.ds(t * RCHUNK + g16 * 16, 16)]
            for l in range(16):
                sv = jnp.broadcast_to(lax.slice(ewv, (l,), (l + 1,)), (16,))
                e = g16 * 16 + l
                for q in range(H // 16):
                    sl = pl.ds(q * 16, 16)
                    bufs[b][e, sl] = bufs[b][e, sl] * sv
            return 0

        lax.fori_loop(0, RCHUNK // 16, grp, 0)

    nblk = jnp.where(c == 0, BLK0, BLK1)

    def block(k, _):
        @pl.when(k > 0)
        def _():
            for b in range(RING):
                wait_scatter(b)
        blk = jnp.where(c == 0, s * BLK0, NS * BLK0 + s * BLK1) + k
        pltpu.sync_copy(src3d.at[blk], isrc)
        pltpu.sync_copy(dst3d.at[blk], idst)
        pltpu.sync_copy(ew.at[pl.ds(blk * BLOCK_E, BLOCK_E)], ewb)
        issue_gather(0, 0)
        issue_gather(1, 1)

        def quad(u, _):
            for b in range(RING):
                t = RING * u + b
                wait_gather(b)
                scale(b, t)
                issue_scatter(b, t)
                bp = (b + 2) % RING
                if b < 2:
                    @pl.when(u > 0)
                    def _():
                        wait_scatter(bp)
                    issue_gather(bp, t + 2)
                else:
                    @pl.when(u < CPB // RING - 1)
                    def _():
                        wait_scatter(bp)
                        issue_gather(bp, t + 2)
            return 0

        lax.fori_loop(0, CPB // RING, quad, 0)
        return 0

    lax.fori_loop(0, nblk, block, 0)
    for b in range(RING):
        wait_scatter(b)

    plsc.subcore_barrier()
    pltpu.sync_copy(acc_sh.at[pl.ds(NPT * s, NPT)],
                    out.at[c, pl.ds(NPT * s, NPT)])


_row_call = pl.kernel(
    _row_body,
    out_type=jax.ShapeDtypeStruct((NC, N_PAD, H), _f32),
    mesh=_MESH,
    scratch_types=[
        pltpu.VMEM_SHARED((N_PAD, H), _f32),
        pltpu.VMEM((CPB, RCHUNK), _i32),
        pltpu.VMEM((CPB, RCHUNK), _i32),
        pltpu.VMEM((BLOCK_E,), _f32),
        pltpu.VMEM((RCHUNK, H), _f32),
        pltpu.VMEM((RCHUNK, H), _f32),
        pltpu.VMEM((RCHUNK, H), _f32),
        pltpu.VMEM((RCHUNK, H), _f32),
        pltpu.SemaphoreType.DMA,
        pltpu.SemaphoreType.DMA,
        pltpu.SemaphoreType.DMA,
        pltpu.SemaphoreType.DMA,
        pltpu.SemaphoreType.DMA,
        pltpu.SemaphoreType.DMA,
        pltpu.SemaphoreType.DMA,
        pltpu.SemaphoreType.DMA,
    ],
)


BLK = 256
GRID = (N + BLK - 1) // BLK


def _tc_stage1_body(degp, x1, x2, x3, w1, w2, w3,
                    h1, h2, h3, dv1, dv2, dv3):
    xs = (x1, x2, x3)
    ws = (w1, w2, w3)
    hs = (h1, h2, h3)
    dvs = (dv1, dv2, dv3)
    for o in range(3):
        deg = degp[o, 0, :] + degp[o, 1, :] + 1.0
        dinv = lax.rsqrt(deg)[:, None]
        h = jnp.dot(xs[o][...], ws[o][...], preferred_element_type=_f32)
        hs[o][...] = h * dinv
        dvs[o][...] = dinv


def _tc_stage1(degp, feats, w1s):
    return pl.pallas_call(
        _tc_stage1_body,
        grid=(GRID,),
        in_specs=[
            pl.BlockSpec((3, NC, BLK), lambda i: (0, 0, i)),
            *[pl.BlockSpec((BLK, D), lambda i: (i, 0)) for _ in range(3)],
            *[pl.BlockSpec((D, H), lambda i: (0, 0)) for _ in range(3)],
        ],
        out_specs=[
            *[pl.BlockSpec((BLK, H), lambda i: (i, 0)) for _ in range(3)],
            *[pl.BlockSpec((BLK, 1), lambda i: (i, 0)) for _ in range(3)],
        ],
        out_shape=[
            *[jax.ShapeDtypeStruct((N, H), _f32) for _ in range(3)],
            *[jax.ShapeDtypeStruct((N, 1), _f32) for _ in range(3)],
        ],
    )(degp, *feats, *w1s)


def _tc_mid_body(a1, a2, a3, p1, p2, p3, v1, v2, v3,
                 b1, b2, b3, w1, w2, w3, o1, o2, o3):
    accs = (a1, a2, a3)
    hps = (p1, p2, p3)
    dvs = (v1, v2, v3)
    bs = (b1, b2, b3)
    ws = (w1, w2, w3)
    outs = (o1, o2, o3)
    for o in range(3):
        acc = accs[o][0] + accs[o][1]
        dinv = dvs[o][...]
        x = jnp.maximum(dinv * (acc + hps[o][...]) + bs[o][...], 0.0)
        outs[o][...] = jnp.dot(x, ws[o][...],
                               preferred_element_type=_f32) * dinv


def _tc_mid(accps, hps, dinvs, bs, w2s):
    return pl.pallas_call(
        _tc_mid_body,
        grid=(GRID,),
        in_specs=[
            *[pl.BlockSpec((NC, BLK, H), lambda i: (0, i, 0)) for _ in range(3)],
            *[pl.BlockSpec((BLK, H), lambda i: (i, 0)) for _ in range(3)],
            *[pl.BlockSpec((BLK, 1), lambda i: (i, 0)) for _ in range(3)],
            *[pl.BlockSpec((1, H), lambda i: (0, 0)) for _ in range(3)],
            *[pl.BlockSpec((H, H), lambda i: (0, 0)) for _ in range(3)],
        ],
        out_specs=[pl.BlockSpec((BLK, H), lambda i: (i, 0)) for _ in range(3)],
        out_shape=[jax.ShapeDtypeStruct((N, H), _f32) for _ in range(3)],
    )(*accps, *hps, *dinvs, *bs, *w2s)


def _tc_final_body(a1, a2, a3, p1, p2, p3, v1, v2, v3, b1, b2, b3,
                   wf, bf, wp, bp, out):
    accs = (a1, a2, a3)
    hps = (p1, p2, p3)
    dvs = (v1, v2, v3)
    bs = (b1, b2, b3)
    wf_all = wf[...]
    csum = jnp.zeros((BLK, H), _f32)
    for o in range(3):
        acc = accs[o][0] + accs[o][1]
        dinv = dvs[o][...]
        x = jnp.maximum(dinv * (acc + hps[o][...]) + bs[o][...], 0.0)
        csum = csum + jnp.dot(x, wf_all[o * H:(o + 1) * H, :],
                              preferred_element_type=_f32)
    fused = jnp.maximum(csum + bf[...], 0.0)
    out[...] = jnp.dot(fused, wp[...], preferred_element_type=_f32) + bp[0, 0]


def _tc_final(accps, hps, dinvs, bs, wf, bf, wp, bp):
    return pl.pallas_call(
        _tc_final_body,
        grid=(GRID,),
        in_specs=[
            *[pl.BlockSpec((NC, BLK, H), lambda i: (0, i, 0)) for _ in range(3)],
            *[pl.BlockSpec((BLK, H), lambda i: (i, 0)) for _ in range(3)],
            *[pl.BlockSpec((BLK, 1), lambda i: (i, 0)) for _ in range(3)],
            *[pl.BlockSpec((1, H), lambda i: (0, 0)) for _ in range(3)],
            pl.BlockSpec((3 * H, H), lambda i: (0, 0)),
            pl.BlockSpec((1, H), lambda i: (0, 0)),
            pl.BlockSpec((H, 1), lambda i: (0, 0)),
            pl.BlockSpec((1, 1), lambda i: (0, 0)),
        ],
        out_specs=pl.BlockSpec((BLK, 1), lambda i: (i, 0)),
        out_shape=jax.ShapeDtypeStruct((N, 1), _f32),
    )(*accps, *hps, *dinvs, *bs, wf, bf, wp, bp)


def _pad_edges(edge_index, edge_weight):
    pad = E_PAD - E
    src = jnp.concatenate([edge_index[0], jnp.zeros((pad,), _i32)])
    dst = jnp.concatenate([edge_index[1], jnp.zeros((pad,), _i32)])
    ew = jnp.concatenate([edge_weight, jnp.zeros((pad,), _f32)])
    return (src.reshape(E_PAD // BLOCK_E, CPB, RCHUNK),
            dst.reshape(E_PAD // BLOCK_E, CPB, RCHUNK),
            dst.reshape(E_PAD // CHUNK, CHUNK), ew)


def kernel(features_o1, features_o2, features_o3,
           edge_index_o1, edge_index_o2, edge_index_o3,
           edge_weight_o1, edge_weight_o2, edge_weight_o3,
           W1_o1, b1_o1, W2_o1, b2_o1,
           W1_o2, b1_o2, W2_o2, b2_o2,
           W1_o3, b1_o3, W2_o3, b2_o3,
           Wf, bf, Wp, bp):
    feats = (features_o1, features_o2, features_o3)
    eis = (edge_index_o1, edge_index_o2, edge_index_o3)
    ews = (edge_weight_o1, edge_weight_o2, edge_weight_o3)
    w1s = (W1_o1, W1_o2, W1_o3)
    b1s = tuple(b.reshape(1, H) for b in (b1_o1, b1_o2, b1_o3))
    w2s = (W2_o1, W2_o2, W2_o3)
    b2s = tuple(b.reshape(1, H) for b in (b2_o1, b2_o2, b2_o3))

    padded = [_pad_edges(ei, ew) for ei, ew in zip(eis, ews)]
    srcs = [p[0] for p in padded]
    dsts = [p[1] for p in padded]
    dst2s = [p[2] for p in padded]
    ewps = [p[3] for p in padded]

    degp = _deg_call(dst2s[0], dst2s[1], dst2s[2], ewps[0], ewps[1], ewps[2])

    h1p_1, h1p_2, h1p_3, dv1, dv2, dv3 = _tc_stage1(degp, feats, w1s)
    h1ps = (h1p_1, h1p_2, h1p_3)
    dinvs = (dv1, dv2, dv3)

    accp1 = [_row_call(h1ps[o], srcs[o], dsts[o], ewps[o]) for o in range(3)]
    h2ps = _tc_mid(accp1, h1ps, dinvs, b1s, w2s)
    accp2 = [_row_call(h2ps[o], srcs[o], dsts[o], ewps[o]) for o in range(3)]

    return _tc_final(accp2, h2ps, dinvs, b2s, Wf,
                     bf.reshape(1, H), Wp, bp.reshape(1, 1))


# speedup vs baseline: 1.0698x; 1.0698x over previous
import functools

import jax
import jax.numpy as jnp
from jax import lax
from jax.experimental import pallas as pl
from jax.experimental.pallas import tpu as pltpu, tpu_sc as plsc

N = 10000
D = 128
H = 128
E = 320000

NC = 2
NS = 16
NW = NC * NS

CHUNK = 128
SUB = 8
BLOCK_E = CHUNK * SUB
EPW_BLOCKS = 10
EPW = BLOCK_E * EPW_BLOCKS
E_PAD = EPW * NW
ROWS_PER_BLOCK = BLOCK_E // CHUNK * SUB // SUB

N_PAD = 10240
NPT = N_PAD // NS

_f32 = jnp.float32
_i32 = jnp.int32

_MESH = plsc.VectorSubcoreMesh(
    core_axis_name="c", subcore_axis_name="s", num_cores=NC, num_subcores=NS)


def _zero_fill(vref, rows, width):
    zero = jnp.zeros((16,), _f32)

    def body(i, _):
        for q in range(width // 16):
            vref[i, pl.ds(q * 16, 16)] = zero
        return 0

    lax.fori_loop(0, rows, body, 0)


def _deg_body(dst1, dst2, dst3, ew1, ew2, ew3, out,
              sh1, sh2, sh3, idx2, ewb, zbuf):
    c = lax.axis_index("c")
    s = lax.axis_index("s")
    w = s * NC + c
    shared = (sh1, sh2, sh3)
    dsts = (dst1, dst2, dst3)
    ews = (ew1, ew2, ew3)

    zero = jnp.zeros((16,), _f32)
    for i in range(40):
        zbuf[pl.ds(16 * i, 16)] = zero
    for o in range(3):
        pltpu.sync_copy(zbuf, shared[o].at[pl.ds(640 * s, 640)])
    plsc.subcore_barrier()

    for o in range(3):
        def chunk(k, _, o=o):
            rb = w * (EPW // CHUNK) + k * SUB
            pltpu.sync_copy(dsts[o].at[pl.ds(rb, SUB)], idx2)
            pltpu.sync_copy(ews[o].at[pl.ds(rb * CHUNK, BLOCK_E)], ewb)
            for j in range(SUB):
                pltpu.sync_copy(ewb.at[pl.ds(j * CHUNK, CHUNK)],
                                shared[o].at[idx2.at[j]], add=True)
            return 0
        lax.fori_loop(0, EPW_BLOCKS, chunk, 0)
    plsc.subcore_barrier()

    for o in range(3):
        pltpu.sync_copy(shared[o].at[pl.ds(640 * s, 640)],
                        out.at[o, c, pl.ds(640 * s, 640)])


_deg_call = pl.kernel(
    _deg_body,
    out_type=jax.ShapeDtypeStruct((3, NC, N_PAD), _f32),
    mesh=_MESH,
    scratch_types=[
        pltpu.VMEM_SHARED((N_PAD,), _f32),
        pltpu.VMEM_SHARED((N_PAD,), _f32),
        pltpu.VMEM_SHARED((N_PAD,), _f32),
        pltpu.VMEM((SUB, CHUNK), _i32),
        pltpu.VMEM((BLOCK_E,), _f32),
        pltpu.VMEM((640,), _f32),
    ],
)


RCHUNK = 64
RING = 4
CPB = BLOCK_E // RCHUNK
BLK0 = 18
BLK1 = 2


def _row_body(hp, src3d, dst3d, ew, out, acc_sh, isrc, idst, ewb,
              b0, b1, b2, b3, gsem0, gsem1, gsem2, gsem3,
              ssem0, ssem1, ssem2, ssem3):
    c = lax.axis_index("c")
    s = lax.axis_index("s")
    w = s * NC + c
    bufs = (b0, b1, b2, b3)
    gsems = (gsem0, gsem1, gsem2, gsem3)
    ssems = (ssem0, ssem1, ssem2, ssem3)

    _zero_fill(b0, RCHUNK, H)
    base = NPT * s
    for k in range(NPT // RCHUNK):
        pltpu.sync_copy(b0, acc_sh.at[pl.ds(base + RCHUNK * k, RCHUNK)])
    plsc.subcore_barrier()

    def issue_gather(b, t):
        pltpu.async_copy(hp.at[isrc.at[t]], bufs[b], gsems[b])

    def wait_gather(b):
        pltpu.make_async_copy(hp.at[isrc.at[0]], bufs[b], gsems[b]).wait()

    def issue_scatter(b, t):
        pltpu.async_copy(bufs[b], acc_sh.at[idst.at[t]], ssems[b], add=True)

    def wait_scatter(b):
        pltpu.make_async_copy(bufs[b], acc_sh.at[idst.at[0]], ssems[b]).wait()

    def scale(b, t):
        def grp(g16, _):
            ewv = ewb[pl.ds(t * RCHUNK + g16 * 16, 16)]
            for l in range(16):
                sv = jnp.broadcast_to(lax.slice(ewv, (l,), (l + 1,)), (16,))
                e = g16 * 16 + l
                for q in range(H // 16):
                    sl = pl.ds(q * 16, 16)
                    bufs[b][e, sl] = bufs[b][e, sl] * sv
            return 0

        lax.fori_loop(0, RCHUNK // 16, grp, 0)

    nblk = jnp.where(c == 0, BLK0, BLK1)

    def block(k, _):
        @pl.when(k > 0)
        def _():
            for b in range(RING):
                wait_scatter(b)
        blk = jnp.where(c == 0, s * BLK0, NS * BLK0 + s * BLK1) + k
        pltpu.sync_copy(src3d.at[blk], isrc)
        pltpu.sync_copy(dst3d.at[blk], idst)
        pltpu.sync_copy(ew.at[pl.ds(blk * BLOCK_E, BLOCK_E)], ewb)
        issue_gather(0, 0)
        issue_gather(1, 1)

        def quad(u, _):
            for b in range(RING):
                t = RING * u + b
                wait_gather(b)
                scale(b, t)
                issue_scatter(b, t)
                bp = (b + 2) % RING
                if b < 2:
                    @pl.when(u > 0)
                    def _():
                        wait_scatter(bp)
                    issue_gather(bp, t + 2)
                else:
                    @pl.when(u < CPB // RING - 1)
                    def _():
                        wait_scatter(bp)
                        issue_gather(bp, t + 2)
            return 0

        lax.fori_loop(0, CPB // RING, quad, 0)
        return 0

    lax.fori_loop(0, nblk, block, 0)
    for b in range(RING):
        wait_scatter(b)

    plsc.subcore_barrier()
    pltpu.sync_copy(acc_sh.at[pl.ds(NPT * s, NPT)],
                    out.at[c, pl.ds(NPT * s, NPT)])


_row_call = pl.kernel(
    _row_body,
    out_type=jax.ShapeDtypeStruct((NC, N_PAD, H), _f32),
    mesh=_MESH,
    scratch_types=[
        pltpu.VMEM_SHARED((N_PAD, H), _f32),
        pltpu.VMEM((CPB, RCHUNK), _i32),
        pltpu.VMEM((CPB, RCHUNK), _i32),
        pltpu.VMEM((BLOCK_E,), _f32),
        pltpu.VMEM((RCHUNK, H), _f32),
        pltpu.VMEM((RCHUNK, H), _f32),
        pltpu.VMEM((RCHUNK, H), _f32),
        pltpu.VMEM((RCHUNK, H), _f32),
        pltpu.SemaphoreType.DMA,
        pltpu.SemaphoreType.DMA,
        pltpu.SemaphoreType.DMA,
        pltpu.SemaphoreType.DMA,
        pltpu.SemaphoreType.DMA,
        pltpu.SemaphoreType.DMA,
        pltpu.SemaphoreType.DMA,
        pltpu.SemaphoreType.DMA,
    ],
)


BLK = 256
GRID = (N + BLK - 1) // BLK


def _tc_stage1_body(degp, x1, x2, x3, w1, w2, w3,
                    h1, h2, h3, dv1, dv2, dv3):
    xs = (x1, x2, x3)
    ws = (w1, w2, w3)
    hs = (h1, h2, h3)
    dvs = (dv1, dv2, dv3)
    for o in range(3):
        deg = degp[o, 0, :] + degp[o, 1, :] + 1.0
        dinv = lax.rsqrt(deg)[:, None]
        h = jnp.dot(xs[o][...], ws[o][...], preferred_element_type=_f32)
        hs[o][...] = h * dinv
        dvs[o][...] = dinv


def _tc_stage1(degp, feats, w1s):
    return pl.pallas_call(
        _tc_stage1_body,
        grid=(GRID,),
        in_specs=[
            pl.BlockSpec((3, NC, BLK), lambda i: (0, 0, i)),
            *[pl.BlockSpec((BLK, D), lambda i: (i, 0)) for _ in range(3)],
            *[pl.BlockSpec((D, H), lambda i: (0, 0)) for _ in range(3)],
        ],
        out_specs=[
            *[pl.BlockSpec((BLK, H), lambda i: (i, 0)) for _ in range(3)],
            *[pl.BlockSpec((BLK, 1), lambda i: (i, 0)) for _ in range(3)],
        ],
        out_shape=[
            *[jax.ShapeDtypeStruct((N, H), _f32) for _ in range(3)],
            *[jax.ShapeDtypeStruct((N, 1), _f32) for _ in range(3)],
        ],
    )(degp, *feats, *w1s)


def _tc_mid_body(a1, a2, a3, p1, p2, p3, v1, v2, v3,
                 b1, b2, b3, w1, w2, w3, o1, o2, o3):
    accs = (a1, a2, a3)
    hps = (p1, p2, p3)
    dvs = (v1, v2, v3)
    bs = (b1, b2, b3)
    ws = (w1, w2, w3)
    outs = (o1, o2, o3)
    for o in range(3):
        acc = accs[o][0] + accs[o][1]
        dinv = dvs[o][...]
        x = jnp.maximum(dinv * (acc + hps[o][...]) + bs[o][...], 0.0)
        outs[o][...] = jnp.dot(x, ws[o][...],
                               preferred_element_type=_f32) * dinv


def _tc_mid(accps, hps, dinvs, bs, w2s):
    return pl.pallas_call(
        _tc_mid_body,
        grid=(GRID,),
        in_specs=[
            *[pl.BlockSpec((NC, BLK, H), lambda i: (0, i, 0)) for _ in range(3)],
            *[pl.BlockSpec((BLK, H), lambda i: (i, 0)) for _ in range(3)],
            *[pl.BlockSpec((BLK, 1), lambda i: (i, 0)) for _ in range(3)],
            *[pl.BlockSpec((1, H), lambda i: (0, 0)) for _ in range(3)],
            *[pl.BlockSpec((H, H), lambda i: (0, 0)) for _ in range(3)],
        ],
        out_specs=[pl.BlockSpec((BLK, H), lambda i: (i, 0)) for _ in range(3)],
        out_shape=[jax.ShapeDtypeStruct((N, H), _f32) for _ in range(3)],
    )(*accps, *hps, *dinvs, *bs, *w2s)


def _tc_final_body(a1, a2, a3, p1, p2, p3, v1, v2, v3, b1, b2, b3,
                   wf, bf, wp, bp, out):
    accs = (a1, a2, a3)
    hps = (p1, p2, p3)
    dvs = (v1, v2, v3)
    bs = (b1, b2, b3)
    wf_all = wf[...]
    csum = jnp.zeros((BLK, H), _f32)
    for o in range(3):
        acc = accs[o][0] + accs[o][1]
        dinv = dvs[o][...]
        x = jnp.maximum(dinv * (acc + hps[o][...]) + bs[o][...], 0.0)
        csum = csum + jnp.dot(x, wf_all[o * H:(o + 1) * H, :],
                              preferred_element_type=_f32)
    fused = jnp.maximum(csum + bf[...], 0.0)
    out[...] = jnp.dot(fused, wp[...], preferred_element_type=_f32) + bp[0, 0]


def _tc_final(accps, hps, dinvs, bs, wf, bf, wp, bp):
    return pl.pallas_call(
        _tc_final_body,
        grid=(GRID,),
        in_specs=[
            *[pl.BlockSpec((NC, BLK, H), lambda i: (0, i, 0)) for _ in range(3)],
            *[pl.BlockSpec((BLK, H), lambda i: (i, 0)) for _ in range(3)],
            *[pl.BlockSpec((BLK, 1), lambda i: (i, 0)) for _ in range(3)],
            *[pl.BlockSpec((1, H), lambda i: (0, 0)) for _ in range(3)],
            pl.BlockSpec((3 * H, H), lambda i: (0, 0)),
            pl.BlockSpec((1, H), lambda i: (0, 0)),
            pl.BlockSpec((H, 1), lambda i: (0, 0)),
            pl.BlockSpec((1, 1), lambda i: (0, 0)),
        ],
        out_specs=pl.BlockSpec((BLK, 1), lambda i: (i, 0)),
        out_shape=jax.ShapeDtypeStruct((N, 1), _f32),
    )(*accps, *hps, *dinvs, *bs, wf, bf, wp, bp)


def _pad_edges(edge_index, edge_weight):
    pad = E_PAD - E
    src = jnp.concatenate([edge_index[0], jnp.zeros((pad,), _i32)])
    dst = jnp.concatenate([edge_index[1], jnp.zeros((pad,), _i32)])
    ew = jnp.concatenate([edge_weight, jnp.zeros((pad,), _f32)])
    return (src.reshape(E_PAD // BLOCK_E, CPB, RCHUNK),
            dst.reshape(E_PAD // BLOCK_E, CPB, RCHUNK),
            dst.reshape(E_PAD // CHUNK, CHUNK), ew)


def kernel(features_o1, features_o2, features_o3,
           edge_index_o1, edge_index_o2, edge_index_o3,
           edge_weight_o1, edge_weight_o2, edge_weight_o3,
           W1_o1, b1_o1, W2_o1, b2_o1,
           W1_o2, b1_o2, W2_o2, b2_o2,
           W1_o3, b1_o3, W2_o3, b2_o3,
           Wf, bf, Wp, bp):
    feats = (features_o1, features_o2, features_o3)
    eis = (edge_index_o1, edge_index_o2, edge_index_o3)
    ews = (edge_weight_o1, edge_weight_o2, edge_weight_o3)
    w1s = (W1_o1, W1_o2, W1_o3)
    b1s = tuple(b.reshape(1, H) for b in (b1_o1, b1_o2, b1_o3))
    w2s = (W2_o1, W2_o2, W2_o3)
    b2s = tuple(b.reshape(1, H) for b in (b2_o1, b2_o2, b2_o3))

    padded = [_pad_edges(ei, ew) for ei, ew in zip(eis, ews)]
    srcs = [p[0] for p in padded]
    dsts = [p[1] for p in padded]
    dst2s = [p[2] for p in padded]
    ewps = [p[3] for p in padded]

    degp = _deg_call(dst2s[0], dst2s[1], dst2s[2], ewps[0], ewps[1], ewps[2])

    h1p_1, h1p_2, h1p_3, dv1, dv2, dv3 = _tc_stage1(degp, feats, w1s)
    h1ps = (h1p_1, h1p_2, h1p_3)
    dinvs = (dv1, dv2, dv3)

    accp1 = [_row_call(h1ps[o], srcs[o], dsts[o], ewps[o]) for o in range(3)]
    h2ps = _tc_mid(accp1, h1ps, dinvs, b1s, w2s)
    accp2 = [_row_call(h2ps[o], srcs[o], dsts[o], ewps[o]) for o in range(3)]

    return _tc_final(accp2, h2ps, dinvs, b2s, Wf,
                     bf.reshape(1, H), Wp, bp.reshape(1, 1))
